# baseline (speedup 1.0000x reference)
import jax
import jax.numpy as jnp
from jax import lax
from jax.experimental import pallas as pl
from jax.experimental.pallas import tpu as pltpu

MAX_LEN = 20
CEMB = 32
H = 128
BT = 512


def _gru_body(spell_ref, len_ref, cemb_ref, wih_ref, whh_ref, bih_ref,
              bhh_ref, out_ref, h_ref):
    lengths = len_ref[...]
    maxlen = jnp.max(lengths)
    cemb = cemb_ref[...].astype(jnp.bfloat16)
    wih = wih_ref[...].astype(jnp.bfloat16)
    whh = whh_ref[...].astype(jnp.bfloat16)
    bih = bih_ref[...]
    bhh = bhh_ref[...]
    bt = out_ref.shape[0]
    iota = lax.broadcasted_iota(jnp.int32, (bt, 128), 1)
    h_ref[...] = jnp.zeros((bt, H), jnp.float32)

    for t in range(MAX_LEN):
        @pl.when(t < maxlen)
        def _step():
            h = h_ref[...]
            c = spell_ref[:, t:t + 1]
            oh = (c == iota).astype(jnp.bfloat16)
            x = jnp.dot(oh, cemb, preferred_element_type=jnp.float32)
            gi = jnp.dot(x.astype(jnp.bfloat16), wih,
                         preferred_element_type=jnp.float32) + bih
            gh = jnp.dot(h.astype(jnp.bfloat16), whh,
                         preferred_element_type=jnp.float32) + bhh
            r = jax.nn.sigmoid(gi[:, :H] + gh[:, :H])
            z = jax.nn.sigmoid(gi[:, H:2 * H] + gh[:, H:2 * H])
            n = jnp.tanh(gi[:, 2 * H:] + r * gh[:, 2 * H:])
            hn = (1.0 - z) * n + z * h
            h_ref[...] = jnp.where(t < lengths, hn, h)

    out_ref[...] = h_ref[...]


def _gru_sorted(s_spell, s_len, char_emb, wihT, whhT, bih2, bhh2):
    B = s_spell.shape[0]
    grid = (B // BT,)
    return pl.pallas_call(
        _gru_body,
        grid=grid,
        in_specs=[
            pl.BlockSpec((BT, MAX_LEN), lambda i: (i, 0)),
            pl.BlockSpec((BT, 1), lambda i: (i, 0)),
            pl.BlockSpec((128, CEMB), lambda i: (0, 0)),
            pl.BlockSpec((CEMB, 3 * H), lambda i: (0, 0)),
            pl.BlockSpec((H, 3 * H), lambda i: (0, 0)),
            pl.BlockSpec((1, 3 * H), lambda i: (0, 0)),
            pl.BlockSpec((1, 3 * H), lambda i: (0, 0)),
        ],
        out_specs=pl.BlockSpec((BT, H), lambda i: (i, 0)),
        out_shape=jax.ShapeDtypeStruct((B, H), jnp.float32),
        scratch_shapes=[pltpu.VMEM((BT, H), jnp.float32)],
    )(s_spell, s_len, char_emb, wihT, whhT, bih2, bhh2)


def kernel(data, spelling_table, char_emb, W_ih, W_hh, b_ih, b_hh):
    rows = spelling_table[data]
    spelling = rows[:, :MAX_LEN]
    lengths = rows[:, MAX_LEN]
    order = jnp.argsort(-lengths)
    s_spell = spelling[order]
    s_len = lengths[order]
    h_sorted = _gru_sorted(s_spell, s_len[:, None], char_emb,
                           W_ih.T, W_hh.T, b_ih[None, :], b_hh[None, :])
    return h_sorted[order]


# speedup vs baseline: 1.0169x; 1.0169x over previous
import jax
import jax.numpy as jnp
from jax import lax
from jax.experimental import pallas as pl
from jax.experimental.pallas import tpu as pltpu
from jax.experimental.pallas import tpu_sc as plsc

MAX_LEN = 20
CEMB = 32
H = 128
B = 4096
BT = 512
NW = 16
CH = B // NW
NV = CH // 16


def _sc_sort_body(len_hbm, iota_hbm, order_hbm,
                  len_v, giota_v, excl_v, cnt_v, all_v, offtab_v, prior_v,
                  rank0, rank1, shared_cnt, order_sh, sem):
    NJ = CH // 128
    rankb = (rank0, rank1)
    wid = lax.axis_index("s")
    base = wid * CH
    i16 = lax.broadcasted_iota(jnp.int32, (16,), 0)
    zeros = jnp.zeros((16,), jnp.int32)

    scps = [pltpu.async_copy(len_hbm.at[pl.ds(base + 128 * j, 128)],
                             len_v.at[pl.ds(128 * j, 128)], sem)
            for j in range(NJ)]
    scps += [pltpu.async_copy(iota_hbm.at[pl.ds(base + 128 * j, 128)],
                              giota_v.at[pl.ds(128 * j, 128)], sem)
             for j in range(NJ)]
    for cp in scps:
        cp.wait()

    cnt_v[pl.ds(0, 16)] = zeros
    cnt_v[pl.ds(16, 16)] = zeros
    c0 = zeros
    c1 = zeros
    for r in range(NV):
        lv = len_v[pl.ds(16 * r, 16)]
        basev = plsc.load_gather(cnt_v, [lv])
        e = zeros
        for v in range(1, MAX_LEN + 1):
            m = lv == v
            incl = jnp.cumsum(m.astype(jnp.int32))
            pop = plsc.all_reduce_population_count(m)
            e = jnp.where(m, incl - 1, e)
            if v < 16:
                c0 = jnp.where(i16 == v, c0 + pop, c0)
            else:
                c1 = jnp.where(i16 == (v - 16), c1 + pop, c1)
        excl_v[pl.ds(16 * r, 16)] = e + basev
        cnt_v[pl.ds(0, 16)] = c0
        cnt_v[pl.ds(16, 16)] = c1

    pltpu.sync_copy(cnt_v.at[pl.ds(0, 32)], shared_cnt.at[wid])
    plsc.subcore_barrier()
    pltpu.sync_copy(shared_cnt, all_v)
    prior_v[pl.ds(0, 16)] = zeros
    prior_v[pl.ds(16, 16)] = zeros
    t0, t1 = zeros, zeros
    for w in range(NW):
        r0 = all_v[w, 0:16]
        r1 = all_v[w, 16:32]
        t0 = t0 + r0
        t1 = t1 + r1

        @pl.when(w < wid)
        def _acc():
            prior_v[pl.ds(0, 16)] = prior_v[pl.ds(0, 16)] + r0
            prior_v[pl.ds(16, 16)] = prior_v[pl.ds(16, 16)] + r1
    rc1 = jnp.cumsum(lax.rev(t1, (0,)))
    s1 = lax.rev(rc1, (0,)) - t1
    cnt_v[pl.ds(0, 16)] = rc1
    tot1 = plsc.load_gather(cnt_v, [jnp.full((16,), 15, jnp.int32)])
    s0 = lax.rev(jnp.cumsum(lax.rev(t0, (0,))), (0,)) - t0 + tot1
    offtab_v[pl.ds(0, 16)] = s0 + prior_v[pl.ds(0, 16)]
    offtab_v[pl.ds(16, 16)] = s1 + prior_v[pl.ds(16, 16)]

    for r in range(NV):
        lv = len_v[pl.ds(16 * r, 16)]
        e = excl_v[pl.ds(16 * r, 16)]
        rk = plsc.load_gather(offtab_v, [lv]) + e
        rankb[r // 8][pl.ds((r % 8) * 16, 16)] = rk
    cps = [pltpu.async_copy(giota_v.at[pl.ds(128 * j, 128)],
                            order_sh.at[rankb[j]], sem)
           for j in range(NJ)]
    for cp in cps:
        cp.wait()
    plsc.subcore_barrier()
    pltpu.sync_copy(order_sh.at[pl.ds(base, CH)],
                    order_hbm.at[pl.ds(base, CH)])


@jax.jit
def _sc_sort(lengths):
    mesh = plsc.VectorSubcoreMesh(core_axis_name="c", subcore_axis_name="s",
                                  num_cores=1)
    return pl.kernel(
        _sc_sort_body,
        mesh=mesh,
        compiler_params=pltpu.CompilerParams(needs_layout_passes=False),
        out_type=jax.ShapeDtypeStruct((B,), jnp.int32),
        scratch_types=[
            pltpu.VMEM((CH,), jnp.int32),
            pltpu.VMEM((CH,), jnp.int32),
            pltpu.VMEM((CH,), jnp.int32),
            pltpu.VMEM((128,), jnp.int32),
            pltpu.VMEM((NW, 32), jnp.int32),
            pltpu.VMEM((128,), jnp.int32),
            pltpu.VMEM((32,), jnp.int32),
            pltpu.VMEM((128,), jnp.int32),
            pltpu.VMEM((128,), jnp.int32),
            pltpu.VMEM_SHARED((NW, 32), jnp.int32),
            pltpu.VMEM_SHARED((B,), jnp.int32),
            pltpu.SemaphoreType.DMA,
        ],
    )(lengths, jnp.arange(B, dtype=jnp.int32))


def _gru_body(stab_ref, cemb_ref, wih_ref, whh_ref, bih_ref,
              bhh_ref, out_ref, h_ref):
    lengths = stab_ref[:, MAX_LEN:MAX_LEN + 1]
    maxlen = jnp.max(lengths)
    cemb = cemb_ref[...].astype(jnp.bfloat16)
    wih = wih_ref[...].astype(jnp.bfloat16)
    whh = whh_ref[...].astype(jnp.bfloat16)
    bih = bih_ref[...]
    bhh = bhh_ref[...]
    bt = out_ref.shape[0]
    iota = lax.broadcasted_iota(jnp.int32, (bt, 128), 1)
    h_ref[...] = jnp.zeros((bt, H), jnp.float32)

    for t in range(MAX_LEN):
        @pl.when(t < maxlen)
        def _step():
            h = h_ref[...]
            c = stab_ref[:, t:t + 1]
            oh = (c == iota).astype(jnp.bfloat16)
            x = jnp.dot(oh, cemb, preferred_element_type=jnp.float32)
            gi = jnp.dot(x.astype(jnp.bfloat16), wih,
                         preferred_element_type=jnp.float32) + bih
            gh = jnp.dot(h.astype(jnp.bfloat16), whh,
                         preferred_element_type=jnp.float32) + bhh
            r = jax.nn.sigmoid(gi[:, :H] + gh[:, :H])
            z = jax.nn.sigmoid(gi[:, H:2 * H] + gh[:, H:2 * H])
            n = jnp.tanh(gi[:, 2 * H:] + r * gh[:, 2 * H:])
            hn = (1.0 - z) * n + z * h
            h_ref[...] = jnp.where(t < lengths, hn, h)

    out_ref[...] = h_ref[...]


def _gru_sorted(s_table, char_emb, wihT, whhT, bih2, bhh2):
    grid = (B // BT,)
    return pl.pallas_call(
        _gru_body,
        grid=grid,
        in_specs=[
            pl.BlockSpec((BT, MAX_LEN + 1), lambda i: (i, 0)),
            pl.BlockSpec((128, CEMB), lambda i: (0, 0)),
            pl.BlockSpec((CEMB, 3 * H), lambda i: (0, 0)),
            pl.BlockSpec((H, 3 * H), lambda i: (0, 0)),
            pl.BlockSpec((1, 3 * H), lambda i: (0, 0)),
            pl.BlockSpec((1, 3 * H), lambda i: (0, 0)),
        ],
        out_specs=pl.BlockSpec((BT, H), lambda i: (i, 0)),
        out_shape=jax.ShapeDtypeStruct((B, H), jnp.float32),
        scratch_shapes=[pltpu.VMEM((BT, H), jnp.float32)],
    )(s_table, char_emb, wihT, whhT, bih2, bhh2)


def kernel(data, spelling_table, char_emb, W_ih, W_hh, b_ih, b_hh):
    rows = spelling_table[data]
    order = _sc_sort(rows[:, MAX_LEN])
    s_rows = rows[order]
    h_sorted = _gru_sorted(s_rows, char_emb, W_ih.T, W_hh.T,
                           b_ih[None, :], b_hh[None, :])
    return h_sorted[order]


# speedup vs baseline: 1.0370x; 1.0198x over previous
import jax
import jax.numpy as jnp
from jax import lax
from jax.experimental import pallas as pl
from jax.experimental.pallas import tpu as pltpu
from jax.experimental.pallas import tpu_sc as plsc

MAX_LEN = 20
CEMB = 32
H = 128
B = 4096
BT = 512
NW = 16
CH = B // NW
NV = CH // 16


def _sc_sort_body(len_hbm, iota_hbm, order_hbm,
                  len_v, giota_v, excl_v, cnt_v, all_v, offtab_v, prior_v,
                  rank0, rank1, shared_cnt, order_sh, sem):
    NJ = CH // 128
    rankb = (rank0, rank1)
    wid = lax.axis_index("s")
    base = wid * CH
    i16 = lax.broadcasted_iota(jnp.int32, (16,), 0)
    zeros = jnp.zeros((16,), jnp.int32)

    scps = [pltpu.async_copy(len_hbm.at[pl.ds(base + 128 * j, 128)],
                             len_v.at[pl.ds(128 * j, 128)], sem)
            for j in range(NJ)]
    scps += [pltpu.async_copy(iota_hbm.at[pl.ds(base + 128 * j, 128)],
                              giota_v.at[pl.ds(128 * j, 128)], sem)
             for j in range(NJ)]
    for cp in scps:
        cp.wait()

    cnt_v[pl.ds(0, 16)] = zeros
    cnt_v[pl.ds(16, 16)] = zeros
    c0 = zeros
    c1 = zeros
    for r in range(NV):
        lv = len_v[pl.ds(16 * r, 16)]
        basev = plsc.load_gather(cnt_v, [lv])
        e = zeros
        for v in range(1, MAX_LEN + 1):
            m = lv == v
            incl = jnp.cumsum(m.astype(jnp.int32))
            pop = plsc.all_reduce_population_count(m)
            e = jnp.where(m, incl - 1, e)
            if v < 16:
                c0 = jnp.where(i16 == v, c0 + pop, c0)
            else:
                c1 = jnp.where(i16 == (v - 16), c1 + pop, c1)
        excl_v[pl.ds(16 * r, 16)] = e + basev
        cnt_v[pl.ds(0, 16)] = c0
        cnt_v[pl.ds(16, 16)] = c1

    pltpu.sync_copy(cnt_v.at[pl.ds(0, 32)], shared_cnt.at[wid])
    plsc.subcore_barrier()
    pltpu.sync_copy(shared_cnt, all_v)
    prior_v[pl.ds(0, 16)] = zeros
    prior_v[pl.ds(16, 16)] = zeros
    t0, t1 = zeros, zeros
    for w in range(NW):
        r0 = all_v[w, 0:16]
        r1 = all_v[w, 16:32]
        t0 = t0 + r0
        t1 = t1 + r1

        @pl.when(w < wid)
        def _acc():
            prior_v[pl.ds(0, 16)] = prior_v[pl.ds(0, 16)] + r0
            prior_v[pl.ds(16, 16)] = prior_v[pl.ds(16, 16)] + r1
    rc1 = jnp.cumsum(lax.rev(t1, (0,)))
    s1 = lax.rev(rc1, (0,)) - t1
    cnt_v[pl.ds(0, 16)] = rc1
    tot1 = plsc.load_gather(cnt_v, [jnp.full((16,), 15, jnp.int32)])
    s0 = lax.rev(jnp.cumsum(lax.rev(t0, (0,))), (0,)) - t0 + tot1
    offtab_v[pl.ds(0, 16)] = s0 + prior_v[pl.ds(0, 16)]
    offtab_v[pl.ds(16, 16)] = s1 + prior_v[pl.ds(16, 16)]

    for r in range(NV):
        lv = len_v[pl.ds(16 * r, 16)]
        e = excl_v[pl.ds(16 * r, 16)]
        rk = plsc.load_gather(offtab_v, [lv]) + e
        rankb[r // 8][pl.ds((r % 8) * 16, 16)] = rk
    cps = [pltpu.async_copy(giota_v.at[pl.ds(128 * j, 128)],
                            order_sh.at[rankb[j]], sem)
           for j in range(NJ)]
    for cp in cps:
        cp.wait()
    plsc.subcore_barrier()
    pltpu.sync_copy(order_sh.at[pl.ds(base, CH)],
                    order_hbm.at[pl.ds(base, CH)])


@jax.jit
def _sc_sort(lengths):
    mesh = plsc.VectorSubcoreMesh(core_axis_name="c", subcore_axis_name="s",
                                  num_cores=1)
    return pl.kernel(
        _sc_sort_body,
        mesh=mesh,
        compiler_params=pltpu.CompilerParams(needs_layout_passes=False),
        out_type=jax.ShapeDtypeStruct((B,), jnp.int32),
        scratch_types=[
            pltpu.VMEM((CH,), jnp.int32),
            pltpu.VMEM((CH,), jnp.int32),
            pltpu.VMEM((CH,), jnp.int32),
            pltpu.VMEM((128,), jnp.int32),
            pltpu.VMEM((NW, 32), jnp.int32),
            pltpu.VMEM((128,), jnp.int32),
            pltpu.VMEM((32,), jnp.int32),
            pltpu.VMEM((128,), jnp.int32),
            pltpu.VMEM((128,), jnp.int32),
            pltpu.VMEM_SHARED((NW, 32), jnp.int32),
            pltpu.VMEM_SHARED((B,), jnp.int32),
            pltpu.SemaphoreType.DMA,
        ],
    )(lengths, jnp.arange(B, dtype=jnp.int32))


def _gru_body(stab_ref, cemb_ref, wih_ref, whh_ref, bih_ref,
              bhh_ref, out_ref, h_ref):
    lengths = stab_ref[:, MAX_LEN:MAX_LEN + 1]
    maxlen = jnp.max(lengths)
    cemb = cemb_ref[...].astype(jnp.bfloat16)
    wih = wih_ref[...].astype(jnp.bfloat16)
    whh = whh_ref[...].astype(jnp.bfloat16)
    bih = bih_ref[...]
    bhh = bhh_ref[...]
    bt = out_ref.shape[0]
    iota = lax.broadcasted_iota(jnp.int32, (bt, 128), 1)
    h_ref[...] = jnp.zeros((bt, H), jnp.float32)

    for t in range(MAX_LEN):
        @pl.when(t < maxlen)
        def _step():
            h = h_ref[...]
            c = stab_ref[:, t:t + 1]
            oh = (c == iota).astype(jnp.bfloat16)
            x = jnp.dot(oh, cemb, preferred_element_type=jnp.float32)
            gi = jnp.dot(x.astype(jnp.bfloat16), wih,
                         preferred_element_type=jnp.float32) + bih
            gh = jnp.dot(h.astype(jnp.bfloat16), whh,
                         preferred_element_type=jnp.float32) + bhh
            rz = jnp.tanh((gi[:, :2 * H] + gh[:, :2 * H]) * 0.5)
            r = 0.5 * rz[:, :H] + 0.5
            z = 0.5 * rz[:, H:] + 0.5
            n = jnp.tanh(gi[:, 2 * H:] + r * gh[:, 2 * H:])
            hn = (1.0 - z) * n + z * h
            h_ref[...] = jnp.where(t < lengths, hn, h)

    out_ref[...] = h_ref[...]


def _gru_sorted(s_table, char_emb, wihT, whhT, bih2, bhh2):
    grid = (B // BT,)
    return pl.pallas_call(
        _gru_body,
        grid=grid,
        in_specs=[
            pl.BlockSpec((BT, MAX_LEN + 1), lambda i: (i, 0)),
            pl.BlockSpec((128, CEMB), lambda i: (0, 0)),
            pl.BlockSpec((CEMB, 3 * H), lambda i: (0, 0)),
            pl.BlockSpec((H, 3 * H), lambda i: (0, 0)),
            pl.BlockSpec((1, 3 * H), lambda i: (0, 0)),
            pl.BlockSpec((1, 3 * H), lambda i: (0, 0)),
        ],
        out_specs=pl.BlockSpec((BT, H), lambda i: (i, 0)),
        out_shape=jax.ShapeDtypeStruct((B, H), jnp.float32),
        scratch_shapes=[pltpu.VMEM((BT, H), jnp.float32)],
    )(s_table, char_emb, wihT, whhT, bih2, bhh2)


def kernel(data, spelling_table, char_emb, W_ih, W_hh, b_ih, b_hh):
    rows = spelling_table[data]
    order = _sc_sort(rows[:, MAX_LEN])
    s_rows = rows[order]
    h_sorted = _gru_sorted(s_rows, char_emb, W_ih.T, W_hh.T,
                           b_ih[None, :], b_hh[None, :])
    return h_sorted[order]


# speedup vs baseline: 1.1519x; 1.1108x over previous
import jax
import jax.numpy as jnp
from jax import lax
from jax.experimental import pallas as pl
from jax.experimental.pallas import tpu as pltpu
from jax.experimental.pallas import tpu_sc as plsc

MAX_LEN = 20
CEMB = 32
H = 128
B = 4096
BT = 1024
NW = 16
CH = B // NW
NV = CH // 16


def _sc_sort_body(len_hbm, iota_hbm, order_hbm,
                  len_v, giota_v, excl_v, cnt_v, all_v, offtab_v, prior_v,
                  rank0, rank1, shared_cnt, order_sh, sem):
    NJ = CH // 128
    rankb = (rank0, rank1)
    wid = lax.axis_index("s")
    base = wid * CH
    i16 = lax.broadcasted_iota(jnp.int32, (16,), 0)
    zeros = jnp.zeros((16,), jnp.int32)

    scps = [pltpu.async_copy(len_hbm.at[pl.ds(base + 128 * j, 128)],
                             len_v.at[pl.ds(128 * j, 128)], sem)
            for j in range(NJ)]
    scps += [pltpu.async_copy(iota_hbm.at[pl.ds(base + 128 * j, 128)],
                              giota_v.at[pl.ds(128 * j, 128)], sem)
             for j in range(NJ)]
    for cp in scps:
        cp.wait()

    cnt_v[pl.ds(0, 16)] = zeros
    cnt_v[pl.ds(16, 16)] = zeros
    c0 = zeros
    c1 = zeros
    for r in range(NV):
        lv = len_v[pl.ds(16 * r, 16)]
        basev = plsc.load_gather(cnt_v, [lv])
        e = zeros
        for v in range(1, MAX_LEN + 1):
            m = lv == v
            incl = jnp.cumsum(m.astype(jnp.int32))
            pop = plsc.all_reduce_population_count(m)
            e = jnp.where(m, incl - 1, e)
            if v < 16:
                c0 = jnp.where(i16 == v, c0 + pop, c0)
            else:
                c1 = jnp.where(i16 == (v - 16), c1 + pop, c1)
        excl_v[pl.ds(16 * r, 16)] = e + basev
        cnt_v[pl.ds(0, 16)] = c0
        cnt_v[pl.ds(16, 16)] = c1

    pltpu.sync_copy(cnt_v.at[pl.ds(0, 32)], shared_cnt.at[wid])
    plsc.subcore_barrier()
    pltpu.sync_copy(shared_cnt, all_v)
    prior_v[pl.ds(0, 16)] = zeros
    prior_v[pl.ds(16, 16)] = zeros
    t0, t1 = zeros, zeros
    for w in range(NW):
        r0 = all_v[w, 0:16]
        r1 = all_v[w, 16:32]
        t0 = t0 + r0
        t1 = t1 + r1

        @pl.when(w < wid)
        def _acc():
            prior_v[pl.ds(0, 16)] = prior_v[pl.ds(0, 16)] + r0
            prior_v[pl.ds(16, 16)] = prior_v[pl.ds(16, 16)] + r1
    rc1 = jnp.cumsum(lax.rev(t1, (0,)))
    s1 = lax.rev(rc1, (0,)) - t1
    cnt_v[pl.ds(0, 16)] = rc1
    tot1 = plsc.load_gather(cnt_v, [jnp.full((16,), 15, jnp.int32)])
    s0 = lax.rev(jnp.cumsum(lax.rev(t0, (0,))), (0,)) - t0 + tot1
    offtab_v[pl.ds(0, 16)] = s0 + prior_v[pl.ds(0, 16)]
    offtab_v[pl.ds(16, 16)] = s1 + prior_v[pl.ds(16, 16)]

    for r in range(NV):
        lv = len_v[pl.ds(16 * r, 16)]
        e = excl_v[pl.ds(16 * r, 16)]
        rk = plsc.load_gather(offtab_v, [lv]) + e
        rankb[r // 8][pl.ds((r % 8) * 16, 16)] = rk
    cps = [pltpu.async_copy(giota_v.at[pl.ds(128 * j, 128)],
                            order_sh.at[rankb[j]], sem)
           for j in range(NJ)]
    for cp in cps:
        cp.wait()
    plsc.subcore_barrier()
    pltpu.sync_copy(order_sh.at[pl.ds(base, CH)],
                    order_hbm.at[pl.ds(base, CH)])


@jax.jit
def _sc_sort(lengths):
    mesh = plsc.VectorSubcoreMesh(core_axis_name="c", subcore_axis_name="s",
                                  num_cores=1)
    return pl.kernel(
        _sc_sort_body,
        mesh=mesh,
        compiler_params=pltpu.CompilerParams(needs_layout_passes=False),
        out_type=jax.ShapeDtypeStruct((B,), jnp.int32),
        scratch_types=[
            pltpu.VMEM((CH,), jnp.int32),
            pltpu.VMEM((CH,), jnp.int32),
            pltpu.VMEM((CH,), jnp.int32),
            pltpu.VMEM((128,), jnp.int32),
            pltpu.VMEM((NW, 32), jnp.int32),
            pltpu.VMEM((128,), jnp.int32),
            pltpu.VMEM((32,), jnp.int32),
            pltpu.VMEM((128,), jnp.int32),
            pltpu.VMEM((128,), jnp.int32),
            pltpu.VMEM_SHARED((NW, 32), jnp.int32),
            pltpu.VMEM_SHARED((B,), jnp.int32),
            pltpu.SemaphoreType.DMA,
        ],
    )(lengths, jnp.arange(B, dtype=jnp.int32))


def _gru_body(stab_ref, cemb_ref, wih_ref, whh_ref, bih_ref,
              bhh_ref, out_ref, h_ref):
    lengths = stab_ref[:, MAX_LEN:MAX_LEN + 1]
    maxlen = jnp.max(lengths)
    cemb = cemb_ref[...].astype(jnp.bfloat16)
    wih = wih_ref[...].astype(jnp.bfloat16)
    whh = whh_ref[...].astype(jnp.bfloat16)
    bih = bih_ref[...]
    bhh = bhh_ref[...]
    bt = out_ref.shape[0]
    iota = lax.broadcasted_iota(jnp.int32, (bt, 128), 1)
    h_ref[...] = jnp.zeros((bt, H), jnp.float32)

    for t in range(MAX_LEN):
        @pl.when(t < maxlen)
        def _step():
            h = h_ref[...]
            c = stab_ref[:, t:t + 1]
            oh = (c == iota).astype(jnp.bfloat16)
            x = jnp.dot(oh, cemb, preferred_element_type=jnp.float32)
            gi = jnp.dot(x.astype(jnp.bfloat16), wih,
                         preferred_element_type=jnp.float32) + bih
            gh = jnp.dot(h.astype(jnp.bfloat16), whh,
                         preferred_element_type=jnp.float32) + bhh
            rz = jnp.tanh((gi[:, :2 * H] + gh[:, :2 * H]) * 0.5)
            r = 0.5 * rz[:, :H] + 0.5
            z = 0.5 * rz[:, H:] + 0.5
            n = jnp.tanh(gi[:, 2 * H:] + r * gh[:, 2 * H:])
            hn = (1.0 - z) * n + z * h
            h_ref[...] = jnp.where(t < lengths, hn, h)

    out_ref[...] = h_ref[...]


def _gru_sorted(s_table, char_emb, wihT, whhT, bih2, bhh2):
    grid = (B // BT,)
    return pl.pallas_call(
        _gru_body,
        grid=grid,
        in_specs=[
            pl.BlockSpec((BT, MAX_LEN + 1), lambda i: (i, 0)),
            pl.BlockSpec((128, CEMB), lambda i: (0, 0)),
            pl.BlockSpec((CEMB, 3 * H), lambda i: (0, 0)),
            pl.BlockSpec((H, 3 * H), lambda i: (0, 0)),
            pl.BlockSpec((1, 3 * H), lambda i: (0, 0)),
            pl.BlockSpec((1, 3 * H), lambda i: (0, 0)),
        ],
        out_specs=pl.BlockSpec((BT, H), lambda i: (i, 0)),
        out_shape=jax.ShapeDtypeStruct((B, H), jnp.float32),
        scratch_shapes=[pltpu.VMEM((BT, H), jnp.float32)],
    )(s_table, char_emb, wihT, whhT, bih2, bhh2)


def kernel(data, spelling_table, char_emb, W_ih, W_hh, b_ih, b_hh):
    rows = spelling_table[data]
    order = _sc_sort(rows[:, MAX_LEN])
    s_rows = rows[order]
    h_sorted = _gru_sorted(s_rows, char_emb, W_ih.T, W_hh.T,
                           b_ih[None, :], b_hh[None, :])
    return h_sorted[order]
